# baseline (speedup 1.0000x reference)
import functools

import jax
import jax.numpy as jnp
from jax import lax
from jax.experimental import pallas as pl
from jax.experimental.pallas import tpu as pltpu
from jax.experimental.pallas import tpu_sc as plsc

_LANES = 16
_NB = 16
_NBUF = 4


@functools.lru_cache(maxsize=None)
def _build_sc_embed(S, B, V, D):
    info = plsc.get_sparse_core_info()
    NC, NS = info.num_cores, info.num_subcores
    NW = NC * NS
    assert S % NW == 0 and B % (_NB * _NBUF) == 0 and D % _LANES == 0
    n_pass = S // NW
    n_chunk = B // _NB
    n_grp = D // _LANES
    mesh = plsc.VectorSubcoreMesh(core_axis_name="c", subcore_axis_name="s")

    @functools.partial(
        pl.kernel,
        mesh=mesh,
        out_type=jax.ShapeDtypeStruct((B, S, D), jnp.float32),
        compiler_params=pltpu.CompilerParams(use_tc_tiling_on_sc=False),
        scratch_types=(
            [pltpu.VMEM((n_pass * B,), jnp.int32)]
            + [pltpu.VMEM((_NB, D), jnp.float32) for _ in range(_NBUF)]
            + [pltpu.VMEM((D,), jnp.float32)]
            + [pltpu.SemaphoreType.DMA for _ in range(2 * _NBUF)]
        ),
    )
    def k(ids_hbm, tok_hbm, pos_hbm, out_hbm, idx_all, r0, r1, r2, r3,
          pos_v, g0, g1, g2, g3, s0, s1, s2, s3):
        rows = (r0, r1, r2, r3)
        gsem = (g0, g1, g2, g3)
        ssem = (s0, s1, s2, s3)
        wid = lax.axis_index("s") * NC + lax.axis_index("c")

        for p in range(n_pass):
            pltpu.sync_copy(ids_hbm.at[wid + NW * p],
                            idx_all.at[pl.ds(p * B, B)])

        for p in range(n_pass):
            s = wid + NW * p
            pltpu.sync_copy(pos_hbm.at[s], pos_v)

            def ivec(kc, p=p):
                return idx_all[pl.ds(p * B + kc * _NB, _NB)]

            def out_slc(kc, s=s):
                return out_hbm.at[pl.ds(kc * _NB, _NB), s]

            pltpu.async_copy(tok_hbm.at[ivec(0)], rows[0], gsem[0])
            pltpu.async_copy(tok_hbm.at[ivec(1)], rows[1], gsem[1])

            def outer(t, carry):
                for b in range(_NBUF):
                    kc = t * _NBUF + b
                    pltpu.make_async_copy(
                        tok_hbm.at[ivec(kc)], rows[b], gsem[b]).wait()

                    n2 = (b + 2) % _NBUF

                    @pl.when(kc >= 2)
                    def _():
                        pltpu.make_async_copy(
                            rows[n2], out_slc(kc - 2), ssem[n2]).wait()

                    @pl.when(kc + 2 < n_chunk)
                    def _():
                        pltpu.async_copy(
                            tok_hbm.at[ivec(kc + 2)], rows[n2], gsem[n2])

                    def col_body(j, b=b):
                        c0 = j * _LANES
                        pv = pos_v[pl.ds(c0, _LANES)]
                        for r in range(_NB):
                            rows[b][r, pl.ds(c0, _LANES)] += pv

                    plsc.parallel_loop(0, n_grp, unroll=4)(col_body)
                    pltpu.async_copy(rows[b], out_slc(kc), ssem[b])
                return carry

            lax.fori_loop(0, n_chunk // _NBUF, outer, 0)

            pltpu.make_async_copy(
                rows[(n_chunk - 2) % _NBUF], out_slc(n_chunk - 2),
                ssem[(n_chunk - 2) % _NBUF]).wait()
            pltpu.make_async_copy(
                rows[(n_chunk - 1) % _NBUF], out_slc(n_chunk - 1),
                ssem[(n_chunk - 1) % _NBUF]).wait()

    return k


def kernel(input_ids, token_table, pos_table):
    if input_ids.ndim == 1:
        input_ids = input_ids[None, :]
    B, S = input_ids.shape
    V, D = token_table.shape
    ids_t = input_ids.astype(jnp.int32).T
    return _build_sc_embed(S, B, V, D)(ids_t, token_table, pos_table)


# speedup vs baseline: 6.7522x; 6.7522x over previous
import functools

import jax
import jax.numpy as jnp
from jax import lax
from jax.experimental import pallas as pl
from jax.experimental.pallas import tpu as pltpu
from jax.experimental.pallas import tpu_sc as plsc

_LANES = 16
_NB = 16
_NBUF = 6


@functools.lru_cache(maxsize=None)
def _build_sc_embed(S, B, V, D):
    info = plsc.get_sparse_core_info()
    NC, NS = info.num_cores, info.num_subcores
    NW = NC * NS
    assert S % NW == 0 and B % _NB == 0 and D % _LANES == 0
    n_pass = S // NW
    n_chunk = B // _NB
    n_peel = n_chunk % _NBUF
    n_main = n_chunk // _NBUF
    assert n_peel >= 2 or n_peel == 0
    n_grp = D // _LANES
    mesh = plsc.VectorSubcoreMesh(core_axis_name="c", subcore_axis_name="s")

    @functools.partial(
        pl.kernel,
        mesh=mesh,
        out_type=jax.ShapeDtypeStruct((B, S, D), jnp.float32),
        scratch_types=(
            [pltpu.VMEM((n_pass * B,), jnp.int32)]
            + [pltpu.VMEM((_NB, D), jnp.float32) for _ in range(_NBUF)]
            + [pltpu.VMEM((D,), jnp.float32)]
            + [pltpu.SemaphoreType.DMA for _ in range(2 * _NBUF)]
        ),
    )
    def k(ids_hbm, tok_hbm, pos_hbm, out_hbm, idx_all, *scr):
        rows = scr[:_NBUF]
        pos_v = scr[_NBUF]
        gsem = scr[_NBUF + 1:2 * _NBUF + 1]
        ssem = scr[2 * _NBUF + 1:]
        wid = lax.axis_index("s") * NC + lax.axis_index("c")

        for p in range(n_pass):
            pltpu.sync_copy(ids_hbm.at[wid + NW * p],
                            idx_all.at[pl.ds(p * B, B)])

        for p in range(n_pass):
            s = wid + NW * p
            pltpu.sync_copy(pos_hbm.at[s], pos_v)

            def ivec(kc, p=p):
                return idx_all[pl.ds(p * B + kc * _NB, _NB)]

            def out_slc(kc, s=s):
                return out_hbm.at[pl.ds(kc * _NB, _NB), s]

            def compute(b):
                def col_body(j, b=b):
                    c0 = j * _LANES
                    pv = pos_v[pl.ds(c0, _LANES)]
                    for r in range(_NB):
                        rows[b][r, pl.ds(c0, _LANES)] += pv

                plsc.parallel_loop(0, n_grp, unroll=4)(col_body)

            def body(kc, b, guard_wait, do_prefetch):
                n2 = (b + 2) % _NBUF
                pltpu.make_async_copy(
                    tok_hbm.at[ivec(kc)], rows[b], gsem[b]).wait()

                def drain():
                    pltpu.make_async_copy(
                        rows[n2], out_slc(kc - (_NBUF - 2)), ssem[n2]).wait()

                if guard_wait:
                    pl.when(kc >= _NBUF - 2)(drain)
                else:
                    drain()
                if do_prefetch:
                    pltpu.async_copy(
                        tok_hbm.at[ivec(kc + 2)], rows[n2], gsem[n2])
                compute(b)
                pltpu.async_copy(rows[b], out_slc(kc), ssem[b])

            pltpu.async_copy(tok_hbm.at[ivec(0)], rows[0], gsem[0])
            pltpu.async_copy(tok_hbm.at[ivec(1)], rows[1], gsem[1])

            def outer(t, carry):
                for b in range(_NBUF):
                    body(t * _NBUF + b, b, guard_wait=True, do_prefetch=True)
                return carry

            lax.fori_loop(0, n_main, outer, 0)

            for kc in range(n_main * _NBUF, n_chunk):
                body(kc, kc % _NBUF, guard_wait=False,
                     do_prefetch=kc + 2 < n_chunk)

            for kc_d in range(n_chunk - (_NBUF - 2), n_chunk):
                pltpu.make_async_copy(
                    rows[kc_d % _NBUF], out_slc(kc_d),
                    ssem[kc_d % _NBUF]).wait()

    return k


def kernel(input_ids, token_table, pos_table):
    if input_ids.ndim == 1:
        input_ids = input_ids[None, :]
    B, S = input_ids.shape
    V, D = token_table.shape
    ids_t = input_ids.astype(jnp.int32).T
    return _build_sc_embed(S, B, V, D)(ids_t, token_table, pos_table)


# speedup vs baseline: 6.9311x; 1.0265x over previous
import functools

import jax
import jax.numpy as jnp
from jax import lax
from jax.experimental import pallas as pl
from jax.experimental.pallas import tpu as pltpu
from jax.experimental.pallas import tpu_sc as plsc

_LANES = 16
_NB = 16
_NBUF = 4


@functools.lru_cache(maxsize=None)
def _build_sc_embed(S, B, V, D):
    info = plsc.get_sparse_core_info()
    NC, NS = info.num_cores, info.num_subcores
    NW = NC * NS
    assert S % NW == 0 and B % _NB == 0 and D % _LANES == 0
    n_pass = S // NW
    n_chunk = B // _NB
    n_total = n_pass * n_chunk
    assert n_total % _NBUF == 0 and (n_chunk & (n_chunk - 1)) == 0
    chunk_shift = n_chunk.bit_length() - 1
    n_grp = D // _LANES
    mesh = plsc.VectorSubcoreMesh(core_axis_name="c", subcore_axis_name="s")

    @functools.partial(
        pl.kernel,
        mesh=mesh,
        out_type=jax.ShapeDtypeStruct((B, S, D), jnp.float32),
        scratch_types=(
            [pltpu.VMEM((n_total * _NB,), jnp.int32)]
            + [pltpu.VMEM((_NB, D), jnp.float32) for _ in range(_NBUF)]
            + [pltpu.VMEM((n_pass * D,), jnp.float32)]
            + [pltpu.SemaphoreType.DMA for _ in range(2 * _NBUF)]
        ),
    )
    def k(ids_hbm, tok_hbm, pos_hbm, out_hbm, idx_all, *scr):
        rows = scr[:_NBUF]
        pos_v = scr[_NBUF]
        gsem = scr[_NBUF + 1:2 * _NBUF + 1]
        ssem = scr[2 * _NBUF + 1:]
        wid = lax.axis_index("s") * NC + lax.axis_index("c")

        for p in range(n_pass):
            pltpu.sync_copy(ids_hbm.at[wid + NW * p],
                            idx_all.at[pl.ds(p * B, B)])
            pltpu.sync_copy(pos_hbm.at[wid + NW * p],
                            pos_v.at[pl.ds(p * D, D)])

        def ivec(kc):
            return idx_all[pl.ds(kc * _NB, _NB)]

        def out_slc(kc):
            p = lax.shift_right_logical(kc, chunk_shift)
            b0 = (kc - lax.shift_left(p, chunk_shift)) * _NB
            return out_hbm.at[pl.ds(b0, _NB), wid + NW * p]

        def body(kc, b):
            n2 = (b + 2) % _NBUF
            pltpu.make_async_copy(
                tok_hbm.at[ivec(kc)], rows[b], gsem[b]).wait()

            @pl.when(kc >= _NBUF - 2)
            def _():
                pltpu.make_async_copy(
                    rows[n2], out_slc(kc - (_NBUF - 2)), ssem[n2]).wait()

            @pl.when(kc + 2 < n_total)
            def _():
                pltpu.async_copy(
                    tok_hbm.at[ivec(kc + 2)], rows[n2], gsem[n2])

            poff = lax.shift_right_logical(kc, chunk_shift) * D

            def col_body(j, b=b, poff=poff):
                c0 = j * _LANES
                pv = pos_v[pl.ds(poff + c0, _LANES)]
                for r in range(_NB):
                    rows[b][r, pl.ds(c0, _LANES)] += pv

            plsc.parallel_loop(0, n_grp, unroll=4)(col_body)
            pltpu.async_copy(rows[b], out_slc(kc), ssem[b])

        pltpu.async_copy(tok_hbm.at[ivec(0)], rows[0], gsem[0])
        pltpu.async_copy(tok_hbm.at[ivec(1)], rows[1], gsem[1])

        def outer(t, carry):
            for b in range(_NBUF):
                body(t * _NBUF + b, b)
            return carry

        lax.fori_loop(0, n_total // _NBUF, outer, 0)

        for kc_d in range(n_total - (_NBUF - 2), n_total):
            pltpu.make_async_copy(
                rows[kc_d % _NBUF], out_slc(kc_d),
                ssem[kc_d % _NBUF]).wait()

    return k


def kernel(input_ids, token_table, pos_table):
    if input_ids.ndim == 1:
        input_ids = input_ids[None, :]
    B, S = input_ids.shape
    V, D = token_table.shape
    ids_t = input_ids.astype(jnp.int32).T
    return _build_sc_embed(S, B, V, D)(ids_t, token_table, pos_table)
